# baseline (speedup 1.0000x reference)
import jax
import jax.numpy as jnp
from jax.experimental import pallas as pl
from jax.experimental.pallas import tpu as pltpu

_EPS = 1e-12
_K_OUT = 64
_K_PAD = 128


def _netvlad_body(x_ref, w_ref, b_ref, c_ref, o_ref):
    x = x_ref[0]

    ssq = jnp.sum(x * x, axis=0, keepdims=True)
    inv = 1.0 / jnp.maximum(jnp.sqrt(ssq), _EPS)
    xn = x * inv

    logits = jax.lax.dot_general(
        w_ref[...], xn, (((1,), (0,)), ((), ())),
        preferred_element_type=jnp.float32)
    logits = logits + b_ref[...]

    m = jnp.max(logits, axis=0, keepdims=True)
    e = jnp.exp(logits - m)
    s = jnp.sum(e, axis=0, keepdims=True)
    a = e / s

    agg = jax.lax.dot_general(
        a, xn, (((1,), (1,)), ((), ())),
        preferred_element_type=jnp.float32)
    asum = jnp.sum(a, axis=1, keepdims=True)
    vlad = agg - asum * c_ref[...]

    v = vlad[:_K_OUT, :]
    rn = jnp.sqrt(jnp.sum(v * v, axis=1, keepdims=True))
    v = v / jnp.maximum(rn, _EPS)

    gsq = jnp.sum(jnp.sum(v * v, axis=1, keepdims=True),
                  axis=0, keepdims=True)
    o_ref[0] = v / jnp.maximum(jnp.sqrt(gsq), _EPS)


def kernel(x, conv_w, conv_b, centroids):
    N, C, H, W = x.shape
    K_all = conv_w.shape[0]
    P = H * W

    xf = x.reshape(N, C, P)
    pad = _K_PAD - K_all
    w_p = jnp.pad(conv_w, ((0, pad), (0, 0)))
    b_p = jnp.pad(conv_b, ((0, pad),), constant_values=-1e30).reshape(_K_PAD, 1)
    c_p = jnp.pad(centroids, ((0, pad), (0, 0)))

    out = pl.pallas_call(
        _netvlad_body,
        grid=(N,),
        in_specs=[
            pl.BlockSpec((1, C, P), lambda n: (n, 0, 0)),
            pl.BlockSpec((_K_PAD, C), lambda n: (0, 0)),
            pl.BlockSpec((_K_PAD, 1), lambda n: (0, 0)),
            pl.BlockSpec((_K_PAD, C), lambda n: (0, 0)),
        ],
        out_specs=pl.BlockSpec((1, _K_OUT, C), lambda n: (n, 0, 0)),
        out_shape=jax.ShapeDtypeStruct((N, _K_OUT, C), jnp.float32),
        compiler_params=pltpu.CompilerParams(
            dimension_semantics=("parallel",),
        ),
    )(xf, w_p, b_p, c_p)

    return out.reshape(N, _K_OUT * C)


# speedup vs baseline: 1.1568x; 1.1568x over previous
import jax
import jax.numpy as jnp
from jax.experimental import pallas as pl
from jax.experimental.pallas import tpu as pltpu

_EPS = 1e-12
_K_OUT = 64
_K_PAD = 128


_B = 4


def _netvlad_body(x_ref, w_ref, b_ref, c_ref, o_ref):
    for i in range(_B):
        _one_image(x_ref.at[i], w_ref, b_ref, c_ref, o_ref.at[i])


def _one_image(x_ref, w_ref, b_ref, c_ref, o_ref):
    x = x_ref[...]

    ssq = jnp.sum(x * x, axis=0, keepdims=True)
    inv = 1.0 / jnp.maximum(jnp.sqrt(ssq), _EPS)
    xn = x * inv

    logits = jax.lax.dot_general(
        w_ref[...], xn, (((1,), (0,)), ((), ())),
        preferred_element_type=jnp.float32)
    logits = logits + b_ref[...]

    m = jnp.max(logits, axis=0, keepdims=True)
    e = jnp.exp(logits - m)
    s = jnp.sum(e, axis=0, keepdims=True)
    a = e / s

    agg = jax.lax.dot_general(
        a, xn, (((1,), (1,)), ((), ())),
        preferred_element_type=jnp.float32)
    asum = jnp.sum(a, axis=1, keepdims=True)
    vlad = agg - asum * c_ref[...]

    v = vlad[:_K_OUT, :]
    rn = jnp.sqrt(jnp.sum(v * v, axis=1, keepdims=True))
    v = v / jnp.maximum(rn, _EPS)

    gsq = jnp.sum(jnp.sum(v * v, axis=1, keepdims=True),
                  axis=0, keepdims=True)
    o_ref[...] = v / jnp.maximum(jnp.sqrt(gsq), _EPS)


def kernel(x, conv_w, conv_b, centroids):
    N, C, H, W = x.shape
    K_all = conv_w.shape[0]
    P = H * W

    xf = x.reshape(N, C, P)
    pad = _K_PAD - K_all
    w_p = jnp.pad(conv_w, ((0, pad), (0, 0)))
    b_p = jnp.pad(conv_b, ((0, pad),), constant_values=-1e30).reshape(_K_PAD, 1)
    c_p = jnp.pad(centroids, ((0, pad), (0, 0)))

    out = pl.pallas_call(
        _netvlad_body,
        grid=(N // _B,),
        in_specs=[
            pl.BlockSpec((_B, C, P), lambda n: (n, 0, 0)),
            pl.BlockSpec((_K_PAD, C), lambda n: (0, 0)),
            pl.BlockSpec((_K_PAD, 1), lambda n: (0, 0)),
            pl.BlockSpec((_K_PAD, C), lambda n: (0, 0)),
        ],
        out_specs=pl.BlockSpec((_B, _K_OUT, C), lambda n: (n, 0, 0)),
        out_shape=jax.ShapeDtypeStruct((N, _K_OUT, C), jnp.float32),
        compiler_params=pltpu.CompilerParams(
            dimension_semantics=("parallel",),
            vmem_limit_bytes=56 * 1024 * 1024,
        ),
    )(xf, w_p, b_p, c_p)

    return out.reshape(N, _K_OUT * C)


# speedup vs baseline: 1.1783x; 1.0185x over previous
import jax
import jax.numpy as jnp
from jax.experimental import pallas as pl
from jax.experimental.pallas import tpu as pltpu

_EPS2 = 1e-24
_K_OUT = 64
_K_PAD = 128
_B = 4


def _netvlad_body(x_ref, w_ref, b_ref, c_ref, o_ref):
    for i in range(_B):
        _one_image(x_ref.at[i], w_ref, b_ref, c_ref, o_ref.at[i])


def _one_image(x_ref, w_ref, b_ref, c_ref, o_ref):
    x = x_ref[...]

    ssq = jnp.sum(x * x, axis=0, keepdims=True)
    inv = jax.lax.rsqrt(jnp.maximum(ssq, _EPS2))

    l0 = jax.lax.dot_general(
        w_ref[...], x, (((1,), (0,)), ((), ())),
        preferred_element_type=jnp.float32)
    logits = l0 * inv + b_ref[...]

    m = jnp.max(logits, axis=0, keepdims=True)
    e = jnp.exp(logits - m)
    s = jnp.sum(e, axis=0, keepdims=True)
    a = e / s

    agg = jax.lax.dot_general(
        a * inv, x, (((1,), (1,)), ((), ())),
        preferred_element_type=jnp.float32)
    asum = jnp.sum(a, axis=1, keepdims=True)
    vlad = agg - asum * c_ref[...]

    v = vlad[:_K_OUT, :]
    rsq = jnp.sum(v * v, axis=1, keepdims=True)
    v = v * jax.lax.rsqrt(jnp.maximum(rsq, _EPS2))

    gsq = jnp.sum(jnp.sum(v * v, axis=1, keepdims=True),
                  axis=0, keepdims=True)
    o_ref[...] = v * jax.lax.rsqrt(jnp.maximum(gsq, _EPS2))


def kernel(x, conv_w, conv_b, centroids):
    N, C, H, W = x.shape
    K_all = conv_w.shape[0]
    P = H * W

    xf = x.reshape(N, C, P)
    pad = _K_PAD - K_all
    w_p = jnp.pad(conv_w, ((0, pad), (0, 0)))
    b_p = jnp.pad(conv_b, ((0, pad),), constant_values=-1e30).reshape(_K_PAD, 1)
    c_p = jnp.pad(centroids, ((0, pad), (0, 0)))

    out = pl.pallas_call(
        _netvlad_body,
        grid=(N // _B,),
        in_specs=[
            pl.BlockSpec((_B, C, P), lambda n: (n, 0, 0)),
            pl.BlockSpec((_K_PAD, C), lambda n: (0, 0)),
            pl.BlockSpec((_K_PAD, 1), lambda n: (0, 0)),
            pl.BlockSpec((_K_PAD, C), lambda n: (0, 0)),
        ],
        out_specs=pl.BlockSpec((_B, _K_OUT, C), lambda n: (n, 0, 0)),
        out_shape=jax.ShapeDtypeStruct((N, _K_OUT, C), jnp.float32),
        compiler_params=pltpu.CompilerParams(
            dimension_semantics=("parallel",),
            vmem_limit_bytes=56 * 1024 * 1024,
        ),
    )(xf, w_p, b_p, c_p)

    return out.reshape(N, _K_OUT * C)
